# baseline (speedup 1.0000x reference)
import functools

import jax
import jax.numpy as jnp
from jax import lax
from jax.experimental import pallas as pl
from jax.experimental.pallas import tpu as pltpu
from jax.experimental.pallas import tpu_sc as plsc

N_NODES = 10000
D_FEAT = 128
N_EDGES = 320000

_NC = 2
_NS = 16
_NW = _NC * _NS
_LANES = 16

_E_PER_W = N_EDGES // _NW
_B_CH = 80
_N_CH = _E_PER_W // _B_CH
_N_G = _B_CH // _LANES

def _sc_dot_kernel(h_hbm, src_hbm, dst_hbm, out_hbm,
                   sidx, didx, outv, tmp,
                   srows0, drows0, srows1, drows1,
                   srows2, drows2, srows3, drows3,
                   sem0, sem1, sem2, sem3):
    wid = lax.axis_index("s") * _NC + lax.axis_index("c")
    base_w = wid * _E_PER_W

    pltpu.sync_copy(src_hbm.at[pl.ds(base_w, _E_PER_W)], sidx)
    pltpu.sync_copy(dst_hbm.at[pl.ds(base_w, _E_PER_W)], didx)

    bufs = ((srows0, drows0, sem0), (srows1, drows1, sem1),
            (srows2, drows2, sem2), (srows3, drows3, sem3))

    def start(ch, slot):
        srows, drows, sem = bufs[slot]
        si = sidx.at[pl.ds(ch * _B_CH, _B_CH)]
        di = didx.at[pl.ds(ch * _B_CH, _B_CH)]
        pltpu.async_copy(h_hbm.at[si], srows, sem)
        pltpu.async_copy(h_hbm.at[di], drows, sem)

    def wait(ch, slot):
        srows, drows, sem = bufs[slot]
        si = sidx.at[pl.ds(ch * _B_CH, _B_CH)]
        di = didx.at[pl.ds(ch * _B_CH, _B_CH)]
        pltpu.make_async_copy(h_hbm.at[si], srows, sem).wait()
        pltpu.make_async_copy(h_hbm.at[di], drows, sem).wait()

    rowid = lax.iota(jnp.int32, _LANES)

    def compute(ch, slot):
        srows, drows, _ = bufs[slot]
        for g in range(_N_G):
            def ebody(e, c):
                base = g * _LANES + e
                ps = []
                for j in range(D_FEAT // _LANES):
                    sv = srows[base, pl.ds(j * _LANES, _LANES)]
                    dv = drows[base, pl.ds(j * _LANES, _LANES)]
                    ps.append(sv * dv)
                while len(ps) > 1:
                    ps = [ps[i] + ps[i + 1] for i in range(0, len(ps), 2)]
                tmp[e, pl.ds(0, _LANES)] = ps[0]
                return c

            lax.fori_loop(0, _LANES, ebody, 0)
            cols = [plsc.load_gather(tmp, [rowid, jnp.full((_LANES,), l, jnp.int32)])
                    for l in range(_LANES)]
            while len(cols) > 1:
                cols = [cols[i] + cols[i + 1] for i in range(0, len(cols), 2)]
            outv[pl.ds(ch * _B_CH + g * _LANES, _LANES)] = cols[0]

    _NBUF = 4
    start(0, 0)
    start(1, 1)
    start(2, 2)

    def quad_body(i, c):
        ch = _NBUF * i
        for b in range(_NBUF):
            start(ch + b + 3, (b + 3) % _NBUF)
            wait(ch + b, b)
            compute(ch + b, b)
        return c

    lax.fori_loop(0, 30, quad_body, 0)
    for ch in range(120, 125):
        b = ch % _NBUF
        if ch + 3 < 125:
            start(ch + 3, (ch + 3) % _NBUF)
        wait(ch, b)
        compute(ch, b)

    pltpu.sync_copy(outv, out_hbm.at[pl.ds(base_w, _E_PER_W)])


@functools.partial(
    pl.kernel,
    mesh=plsc.VectorSubcoreMesh(core_axis_name="c", subcore_axis_name="s"),
    out_type=jax.ShapeDtypeStruct((N_EDGES,), jnp.float32),
    compiler_params=pltpu.CompilerParams(needs_layout_passes=False),
    scratch_types=[
        pltpu.VMEM((_E_PER_W,), jnp.int32),
        pltpu.VMEM((_E_PER_W,), jnp.int32),
        pltpu.VMEM((_E_PER_W,), jnp.float32),
        pltpu.VMEM((_LANES, _LANES + 1), jnp.float32),
        pltpu.VMEM((_B_CH, D_FEAT), jnp.float32),
        pltpu.VMEM((_B_CH, D_FEAT), jnp.float32),
        pltpu.VMEM((_B_CH, D_FEAT), jnp.float32),
        pltpu.VMEM((_B_CH, D_FEAT), jnp.float32),
        pltpu.VMEM((_B_CH, D_FEAT), jnp.float32),
        pltpu.VMEM((_B_CH, D_FEAT), jnp.float32),
        pltpu.VMEM((_B_CH, D_FEAT), jnp.float32),
        pltpu.VMEM((_B_CH, D_FEAT), jnp.float32),
        pltpu.SemaphoreType.DMA,
        pltpu.SemaphoreType.DMA,
        pltpu.SemaphoreType.DMA,
        pltpu.SemaphoreType.DMA,
    ],
)
def _dot_predictor(h_hbm, src_hbm, dst_hbm, out_hbm,
                   sidx, didx, outv, tmp,
                   srows0, drows0, srows1, drows1,
                   srows2, drows2, srows3, drows3,
                   sem0, sem1, sem2, sem3):
    _sc_dot_kernel(h_hbm, src_hbm, dst_hbm, out_hbm,
                   sidx, didx, outv, tmp,
                   srows0, drows0, srows1, drows1,
                   srows2, drows2, srows3, drows3,
                   sem0, sem1, sem2, sem3)


def kernel(h, edge_index):
    src = edge_index[0]
    dst = edge_index[1]
    return _dot_predictor(h, src, dst)


# speedup vs baseline: 1.1692x; 1.1692x over previous
import functools

import jax
import jax.numpy as jnp
from jax import lax
from jax.experimental import pallas as pl
from jax.experimental.pallas import tpu as pltpu
from jax.experimental.pallas import tpu_sc as plsc

N_NODES = 10000
D_FEAT = 128
N_EDGES = 320000

_NC = 2
_NS = 16
_NW = _NC * _NS
_LANES = 16

_E_PER_W = N_EDGES // _NW
_B_CH = 40
_N_CH = _E_PER_W // _B_CH
_N_G = _B_CH // _LANES + (1 if _B_CH % _LANES else 0)
_NBUF = 3


def _sc_dot_kernel(h_hbm, src_hbm, dst_hbm, out_hbm,
                   hs, outv, tmp, bufs):
    wid = lax.axis_index("s") * _NC + lax.axis_index("c")
    sid = lax.axis_index("s")
    base_w = wid * _E_PER_W

    @pl.when(sid < _NS - 1)
    def _():
        r0 = sid * 624
        pltpu.sync_copy(h_hbm.at[pl.ds(r0, 624)], hs.at[pl.ds(r0, 624)])

    @pl.when(sid == _NS - 1)
    def _():
        pltpu.sync_copy(h_hbm.at[pl.ds(9360, 640)], hs.at[pl.ds(9360, 640)])

    plsc.subcore_barrier()

    def start_idx(ch, slot):
        sidx, didx, srows, drows, isem, rsem = bufs[slot]
        base = base_w + ch * _B_CH
        pltpu.async_copy(src_hbm.at[pl.ds(base, _B_CH)], sidx, isem)
        pltpu.async_copy(dst_hbm.at[pl.ds(base, _B_CH)], didx, isem)

    def start_rows(ch, slot):
        sidx, didx, srows, drows, isem, rsem = bufs[slot]
        base = base_w + ch * _B_CH
        pltpu.make_async_copy(src_hbm.at[pl.ds(base, _B_CH)], sidx, isem).wait()
        pltpu.make_async_copy(dst_hbm.at[pl.ds(base, _B_CH)], didx, isem).wait()
        pltpu.async_copy(hs.at[sidx], srows, rsem)
        pltpu.async_copy(hs.at[didx], drows, rsem)

    def wait_rows(slot):
        sidx, didx, srows, drows, isem, rsem = bufs[slot]
        pltpu.make_async_copy(hs.at[sidx], srows, rsem).wait()
        pltpu.make_async_copy(hs.at[didx], drows, rsem).wait()

    rowid = lax.iota(jnp.int32, _LANES)

    def compute(ch, slot):
        _, _, srows, drows, _, _ = bufs[slot]
        for g in range(_B_CH // _LANES):
            def ebody(e, c):
                base = g * _LANES + e
                ps = []
                for j in range(D_FEAT // _LANES):
                    sv = srows[base, pl.ds(j * _LANES, _LANES)]
                    dv = drows[base, pl.ds(j * _LANES, _LANES)]
                    ps.append(sv * dv)
                while len(ps) > 1:
                    ps = [ps[i] + ps[i + 1] for i in range(0, len(ps), 2)]
                tmp[e, pl.ds(0, _LANES)] = ps[0]
                return c

            lax.fori_loop(0, _LANES, ebody, 0)
            cols = [plsc.load_gather(
                        tmp, [rowid, jnp.full((_LANES,), l, jnp.int32)])
                    for l in range(_LANES)]
            while len(cols) > 1:
                cols = [cols[i] + cols[i + 1] for i in range(0, len(cols), 2)]
            outv[pl.ds(ch * _B_CH + g * _LANES, _LANES)] = cols[0]

    for k in range(_NBUF - 1):
        start_idx(k, k)
    start_rows(0, 0)

    _MAIN = (_N_CH - (_NBUF - 1)) // _NBUF

    def ring_body(i, c):
        ch = _NBUF * i
        for b in range(_NBUF):
            n = ch + b
            start_idx(n + _NBUF - 1, (b + _NBUF - 1) % _NBUF)
            start_rows(n + 1, (b + 1) % _NBUF)
            wait_rows(b)
            compute(n, b)
        return c

    lax.fori_loop(0, _MAIN, ring_body, 0)
    for n in range(_MAIN * _NBUF, _N_CH):
        b = n % _NBUF
        if n + _NBUF - 1 < _N_CH:
            start_idx(n + _NBUF - 1, (n + _NBUF - 1) % _NBUF)
        if n + 1 < _N_CH:
            start_rows(n + 1, (n + 1) % _NBUF)
        wait_rows(b)
        compute(n, b)

    pltpu.sync_copy(outv, out_hbm.at[pl.ds(base_w, _E_PER_W)])


@functools.partial(
    pl.kernel,
    mesh=plsc.VectorSubcoreMesh(core_axis_name="c", subcore_axis_name="s"),
    out_type=jax.ShapeDtypeStruct((N_EDGES,), jnp.float32),
    compiler_params=pltpu.CompilerParams(needs_layout_passes=False),
    scratch_types=[
        pltpu.VMEM_SHARED((N_NODES, D_FEAT), jnp.float32),
        pltpu.VMEM((_E_PER_W,), jnp.float32),
        pltpu.VMEM((_LANES, _LANES + 1), jnp.float32),
    ] + [
        t
        for _ in range(_NBUF)
        for t in (pltpu.VMEM((_B_CH,), jnp.int32),
                  pltpu.VMEM((_B_CH,), jnp.int32),
                  pltpu.VMEM((_B_CH, D_FEAT), jnp.float32),
                  pltpu.VMEM((_B_CH, D_FEAT), jnp.float32),
                  pltpu.SemaphoreType.DMA,
                  pltpu.SemaphoreType.DMA)
    ],
)
def _dot_predictor(h_hbm, src_hbm, dst_hbm, out_hbm,
                   hs, outv, tmp, *flat_bufs):
    bufs = tuple(tuple(flat_bufs[i * 6:(i + 1) * 6]) for i in range(_NBUF))
    _sc_dot_kernel(h_hbm, src_hbm, dst_hbm, out_hbm, hs, outv, tmp, bufs)


def kernel(h, edge_index):
    src = edge_index[0]
    dst = edge_index[1]
    return _dot_predictor(h, src, dst)
